# baseline (speedup 1.0000x reference)
import jax
import jax.numpy as jnp
from jax.experimental import pallas as pl

_NBLK = 512


def _pass1(u_ref, k_ref, uf_ref, kf_ref, w0_ref, b0_ref,
           y0_ref, s_ref, ss_ref):
    b = pl.program_id(0)
    i = pl.program_id(1)

    u = u_ref[0]
    kt = k_ref[0]
    un = jnp.sum(u * u, axis=1, keepdims=True)
    kn = jnp.sum(kt * kt, axis=0, keepdims=True)
    cross = jax.lax.dot_general(u, kt, (((1,), (0,)), ((), ())),
                                preferred_element_type=jnp.float32,
                                precision=jax.lax.Precision.HIGHEST)
    d2 = jnp.maximum((un + kn) - 2.0 * cross, 1e-30)

    inf = jnp.float32(jnp.inf)
    m1 = jnp.min(d2, axis=1, keepdims=True)
    c1 = d2 <= m1
    m2 = jnp.min(jnp.where(c1, inf, d2), axis=1, keepdims=True)
    c2 = d2 <= m2
    m3 = jnp.min(jnp.where(c2, inf, d2), axis=1, keepdims=True)
    c3 = d2 <= m3

    r1 = jax.lax.rsqrt(m1)
    r2 = jax.lax.rsqrt(m2)
    r3 = jax.lax.rsqrt(m3)
    rnorm = 1.0 / (r1 + r2 + r3)
    wint = jnp.where(c3, jax.lax.rsqrt(d2) * rnorm, 0.0)

    kf = kf_ref[0]
    interp = jax.lax.dot_general(kf, wint, (((1,), (1,)), ((), ())),
                                 preferred_element_type=jnp.float32, precision=jax.lax.Precision.DEFAULT)
    x = jnp.concatenate([interp, uf_ref[0]], axis=0)
    y0 = jax.lax.dot_general(w0_ref[...], x, (((1,), (0,)), ((), ())),
                             preferred_element_type=jnp.float32, precision=jax.lax.Precision.DEFAULT)
    y0 = y0 + b0_ref[...]
    y0_ref[0] = y0.astype(jnp.bfloat16)

    @pl.when((b == 0) & (i == 0))
    def _():
        s_ref[...] = jnp.zeros_like(s_ref)
        ss_ref[...] = jnp.zeros_like(ss_ref)
    s_ref[...] += jnp.sum(y0, axis=1, keepdims=True).T
    ss_ref[...] += jnp.sum(y0 * y0, axis=1, keepdims=True).T


def _pass2(y0_ref, w1_ref, b1_ref, a0_ref, d0_ref,
           y1_ref, s_ref, ss_ref):
    b = pl.program_id(0)
    i = pl.program_id(1)
    z = jnp.maximum(y0_ref[0].astype(jnp.float32) * a0_ref[...] + d0_ref[...], 0.0)
    y1 = jax.lax.dot_general(w1_ref[...], z, (((1,), (0,)), ((), ())),
                             preferred_element_type=jnp.float32, precision=jax.lax.Precision.DEFAULT)
    y1 = y1 + b1_ref[...]
    y1_ref[0] = y1.astype(jnp.bfloat16)

    @pl.when((b == 0) & (i == 0))
    def _():
        s_ref[...] = jnp.zeros_like(s_ref)
        ss_ref[...] = jnp.zeros_like(ss_ref)
    s_ref[...] += jnp.sum(y1, axis=1, keepdims=True).T
    ss_ref[...] += jnp.sum(y1 * y1, axis=1, keepdims=True).T


def _pass3(y1_ref, a1_ref, d1_ref, out_ref):
    y1 = y1_ref[0].astype(jnp.float32)
    out_ref[0] = jnp.maximum(y1 * a1_ref[...] + d1_ref[...], 0.0)


def kernel(unknown, known, unknow_feats, known_feats,
           W0, b0, g0, beta0, W1, b1, g1, beta1):
    B, N, _ = unknown.shape
    M = known.shape[1]
    C1 = unknow_feats.shape[1]
    C2 = known_feats.shape[1]
    C0 = W0.shape[0]
    C3 = W1.shape[0]
    nblk = _NBLK if N % _NBLK == 0 else N
    nb = N // nblk
    cnt = B * N

    f32 = jnp.float32
    y0, s0, ss0 = pl.pallas_call(
        _pass1,
        grid=(B, nb),
        in_specs=[
            pl.BlockSpec((1, nblk, 3), lambda b, i: (b, i, 0)),
            pl.BlockSpec((1, 3, M), lambda b, i: (b, 0, 0)),
            pl.BlockSpec((1, C1, nblk), lambda b, i: (b, 0, i)),
            pl.BlockSpec((1, C2, M), lambda b, i: (b, 0, 0)),
            pl.BlockSpec((C0, C1 + C2), lambda b, i: (0, 0)),
            pl.BlockSpec((C0, 1), lambda b, i: (0, 0)),
        ],
        out_specs=[
            pl.BlockSpec((1, C0, nblk), lambda b, i: (b, 0, i)),
            pl.BlockSpec((1, C0), lambda b, i: (0, 0)),
            pl.BlockSpec((1, C0), lambda b, i: (0, 0)),
        ],
        out_shape=[
            jax.ShapeDtypeStruct((B, C0, N), jnp.bfloat16),
            jax.ShapeDtypeStruct((1, C0), f32),
            jax.ShapeDtypeStruct((1, C0), f32),
        ],
    )(unknown, known.transpose(0, 2, 1), unknow_feats, known_feats,
      W0, b0.reshape(C0, 1))

    mean0 = s0[0] / cnt
    var0 = ss0[0] / cnt - mean0 * mean0
    a0 = g0 / jnp.sqrt(var0 + 1e-5)
    d0 = beta0 - a0 * mean0

    y1, s1, ss1 = pl.pallas_call(
        _pass2,
        grid=(B, nb),
        in_specs=[
            pl.BlockSpec((1, C0, nblk), lambda b, i: (b, 0, i)),
            pl.BlockSpec((C3, C0), lambda b, i: (0, 0)),
            pl.BlockSpec((C3, 1), lambda b, i: (0, 0)),
            pl.BlockSpec((C0, 1), lambda b, i: (0, 0)),
            pl.BlockSpec((C0, 1), lambda b, i: (0, 0)),
        ],
        out_specs=[
            pl.BlockSpec((1, C3, nblk), lambda b, i: (b, 0, i)),
            pl.BlockSpec((1, C3), lambda b, i: (0, 0)),
            pl.BlockSpec((1, C3), lambda b, i: (0, 0)),
        ],
        out_shape=[
            jax.ShapeDtypeStruct((B, C3, N), jnp.bfloat16),
            jax.ShapeDtypeStruct((1, C3), f32),
            jax.ShapeDtypeStruct((1, C3), f32),
        ],
    )(y0, W1, b1.reshape(C3, 1), a0.reshape(C0, 1), d0.reshape(C0, 1))

    mean1 = s1[0] / cnt
    var1 = ss1[0] / cnt - mean1 * mean1
    a1 = g1 / jnp.sqrt(var1 + 1e-5)
    d1 = beta1 - a1 * mean1

    out = pl.pallas_call(
        _pass3,
        grid=(B, nb),
        in_specs=[
            pl.BlockSpec((1, C3, nblk), lambda b, i: (b, 0, i)),
            pl.BlockSpec((C3, 1), lambda b, i: (0, 0)),
            pl.BlockSpec((C3, 1), lambda b, i: (0, 0)),
        ],
        out_specs=pl.BlockSpec((1, C3, nblk), lambda b, i: (b, 0, i)),
        out_shape=jax.ShapeDtypeStruct((B, C3, N), f32),
    )(y1, a1.reshape(C3, 1), d1.reshape(C3, 1))

    return out


# speedup vs baseline: 1.2549x; 1.2549x over previous
import jax
import jax.numpy as jnp
from jax.experimental import pallas as pl

_NBLK = 512


def _pass1(u_ref, k_ref, uf_ref, kf_ref, w0_ref, b0_ref,
           y0_ref, s_ref, ss_ref):
    b = pl.program_id(0)
    i = pl.program_id(1)

    u = u_ref[0]
    kt = k_ref[0]
    du0 = u[:, 0:1] - kt[0:1, :]
    du1 = u[:, 1:2] - kt[1:2, :]
    du2 = u[:, 2:3] - kt[2:3, :]
    d2 = du0 * du0 + du1 * du1 + du2 * du2

    inf = jnp.float32(jnp.inf)
    m1 = jnp.min(d2, axis=1, keepdims=True)
    c1 = d2 <= m1
    m2 = jnp.min(jnp.where(c1, inf, d2), axis=1, keepdims=True)
    c2 = d2 <= m2
    m3 = jnp.min(jnp.where(c2, inf, d2), axis=1, keepdims=True)
    c3 = d2 <= m3

    r1 = jax.lax.rsqrt(m1)
    r2 = jax.lax.rsqrt(m2)
    r3 = jax.lax.rsqrt(m3)
    rnorm = 1.0 / (r1 + r2 + r3)
    wint = jnp.where(c3, jax.lax.rsqrt(d2) * rnorm, 0.0)

    kf = kf_ref[0]
    interp = jax.lax.dot_general(kf, wint, (((1,), (1,)), ((), ())),
                                 preferred_element_type=jnp.float32, precision=jax.lax.Precision.DEFAULT)
    x = jnp.concatenate([interp, uf_ref[0]], axis=0)
    y0 = jax.lax.dot_general(w0_ref[...], x, (((1,), (0,)), ((), ())),
                             preferred_element_type=jnp.float32, precision=jax.lax.Precision.DEFAULT)
    y0 = y0 + b0_ref[...]
    y0_ref[0] = y0.astype(jnp.bfloat16)

    @pl.when((b == 0) & (i == 0))
    def _():
        s_ref[...] = jnp.zeros_like(s_ref)
        ss_ref[...] = jnp.zeros_like(ss_ref)
    s_ref[...] += jnp.sum(y0, axis=1, keepdims=True).T
    ss_ref[...] += jnp.sum(y0 * y0, axis=1, keepdims=True).T


def _pass2(y0_ref, w1_ref, b1_ref, a0_ref, d0_ref,
           y1_ref, s_ref, ss_ref):
    b = pl.program_id(0)
    i = pl.program_id(1)
    z = jnp.maximum(y0_ref[0].astype(jnp.float32) * a0_ref[...] + d0_ref[...], 0.0)
    y1 = jax.lax.dot_general(w1_ref[...], z, (((1,), (0,)), ((), ())),
                             preferred_element_type=jnp.float32, precision=jax.lax.Precision.DEFAULT)
    y1 = y1 + b1_ref[...]
    y1_ref[0] = y1.astype(jnp.bfloat16)

    @pl.when((b == 0) & (i == 0))
    def _():
        s_ref[...] = jnp.zeros_like(s_ref)
        ss_ref[...] = jnp.zeros_like(ss_ref)
    s_ref[...] += jnp.sum(y1, axis=1, keepdims=True).T
    ss_ref[...] += jnp.sum(y1 * y1, axis=1, keepdims=True).T


def _pass3(y1_ref, a1_ref, d1_ref, out_ref):
    y1 = y1_ref[0].astype(jnp.float32)
    out_ref[0] = jnp.maximum(y1 * a1_ref[...] + d1_ref[...], 0.0)


def kernel(unknown, known, unknow_feats, known_feats,
           W0, b0, g0, beta0, W1, b1, g1, beta1):
    B, N, _ = unknown.shape
    M = known.shape[1]
    C1 = unknow_feats.shape[1]
    C2 = known_feats.shape[1]
    C0 = W0.shape[0]
    C3 = W1.shape[0]
    nblk = _NBLK if N % _NBLK == 0 else N
    nb = N // nblk
    cnt = B * N

    f32 = jnp.float32
    y0, s0, ss0 = pl.pallas_call(
        _pass1,
        grid=(B, nb),
        in_specs=[
            pl.BlockSpec((1, nblk, 3), lambda b, i: (b, i, 0)),
            pl.BlockSpec((1, 3, M), lambda b, i: (b, 0, 0)),
            pl.BlockSpec((1, C1, nblk), lambda b, i: (b, 0, i)),
            pl.BlockSpec((1, C2, M), lambda b, i: (b, 0, 0)),
            pl.BlockSpec((C0, C1 + C2), lambda b, i: (0, 0)),
            pl.BlockSpec((C0, 1), lambda b, i: (0, 0)),
        ],
        out_specs=[
            pl.BlockSpec((1, C0, nblk), lambda b, i: (b, 0, i)),
            pl.BlockSpec((1, C0), lambda b, i: (0, 0)),
            pl.BlockSpec((1, C0), lambda b, i: (0, 0)),
        ],
        out_shape=[
            jax.ShapeDtypeStruct((B, C0, N), jnp.bfloat16),
            jax.ShapeDtypeStruct((1, C0), f32),
            jax.ShapeDtypeStruct((1, C0), f32),
        ],
    )(unknown, known.transpose(0, 2, 1), unknow_feats, known_feats,
      W0, b0.reshape(C0, 1))

    mean0 = s0[0] / cnt
    var0 = ss0[0] / cnt - mean0 * mean0
    a0 = g0 / jnp.sqrt(var0 + 1e-5)
    d0 = beta0 - a0 * mean0

    y1, s1, ss1 = pl.pallas_call(
        _pass2,
        grid=(B, nb),
        in_specs=[
            pl.BlockSpec((1, C0, nblk), lambda b, i: (b, 0, i)),
            pl.BlockSpec((C3, C0), lambda b, i: (0, 0)),
            pl.BlockSpec((C3, 1), lambda b, i: (0, 0)),
            pl.BlockSpec((C0, 1), lambda b, i: (0, 0)),
            pl.BlockSpec((C0, 1), lambda b, i: (0, 0)),
        ],
        out_specs=[
            pl.BlockSpec((1, C3, nblk), lambda b, i: (b, 0, i)),
            pl.BlockSpec((1, C3), lambda b, i: (0, 0)),
            pl.BlockSpec((1, C3), lambda b, i: (0, 0)),
        ],
        out_shape=[
            jax.ShapeDtypeStruct((B, C3, N), jnp.bfloat16),
            jax.ShapeDtypeStruct((1, C3), f32),
            jax.ShapeDtypeStruct((1, C3), f32),
        ],
    )(y0, W1, b1.reshape(C3, 1), a0.reshape(C0, 1), d0.reshape(C0, 1))

    mean1 = s1[0] / cnt
    var1 = ss1[0] / cnt - mean1 * mean1
    a1 = g1 / jnp.sqrt(var1 + 1e-5)
    d1 = beta1 - a1 * mean1

    out = pl.pallas_call(
        _pass3,
        grid=(B, nb),
        in_specs=[
            pl.BlockSpec((1, C3, nblk), lambda b, i: (b, 0, i)),
            pl.BlockSpec((C3, 1), lambda b, i: (0, 0)),
            pl.BlockSpec((C3, 1), lambda b, i: (0, 0)),
        ],
        out_specs=pl.BlockSpec((1, C3, nblk), lambda b, i: (b, 0, i)),
        out_shape=jax.ShapeDtypeStruct((B, C3, N), f32),
    )(y1, a1.reshape(C3, 1), d1.reshape(C3, 1))

    return out


# speedup vs baseline: 1.6727x; 1.3329x over previous
import jax
import jax.numpy as jnp
from jax.experimental import pallas as pl

_NBLK = 1024


def _pass1(u_ref, k_ref, uf_ref, kf_ref, w0_ref, b0_ref,
           y0_ref, s_ref, ss_ref):
    b = pl.program_id(0)
    i = pl.program_id(1)

    u = u_ref[0]
    kt = k_ref[0]
    du0 = u[:, 0:1] - kt[0:1, :]
    du1 = u[:, 1:2] - kt[1:2, :]
    du2 = u[:, 2:3] - kt[2:3, :]
    d2 = du0 * du0 + du1 * du1 + du2 * du2

    inf = jnp.float32(jnp.inf)
    m1 = jnp.min(d2, axis=1, keepdims=True)
    c1 = d2 <= m1
    m2 = jnp.min(jnp.where(c1, inf, d2), axis=1, keepdims=True)
    c2 = d2 <= m2
    m3 = jnp.min(jnp.where(c2, inf, d2), axis=1, keepdims=True)
    c3 = d2 <= m3

    r1 = jax.lax.rsqrt(m1)
    r2 = jax.lax.rsqrt(m2)
    r3 = jax.lax.rsqrt(m3)
    rnorm = 1.0 / (r1 + r2 + r3)
    wint = jnp.where(c3, jax.lax.rsqrt(d2) * rnorm, 0.0)

    kf = kf_ref[0]
    C2 = kf.shape[0]
    interp = jax.lax.dot_general(kf, wint, (((1,), (1,)), ((), ())),
                                 preferred_element_type=jnp.float32, precision=jax.lax.Precision.DEFAULT)
    y0 = jax.lax.dot_general(w0_ref[:, :C2], interp, (((1,), (0,)), ((), ())),
                             preferred_element_type=jnp.float32, precision=jax.lax.Precision.DEFAULT)
    y0 = y0 + jax.lax.dot_general(w0_ref[:, C2:], uf_ref[0], (((1,), (0,)), ((), ())),
                                  preferred_element_type=jnp.float32, precision=jax.lax.Precision.DEFAULT)
    y0 = y0 + b0_ref[...]
    y0_ref[0] = y0.astype(jnp.bfloat16)

    @pl.when((b == 0) & (i == 0))
    def _():
        s_ref[...] = jnp.zeros_like(s_ref)
        ss_ref[...] = jnp.zeros_like(ss_ref)
    s_ref[...] += jnp.sum(y0, axis=1, keepdims=True).T
    ss_ref[...] += jnp.sum(y0 * y0, axis=1, keepdims=True).T


def _pass2(y0_ref, w1_ref, b1_ref, a0_ref, d0_ref,
           y1_ref, s_ref, ss_ref):
    b = pl.program_id(0)
    i = pl.program_id(1)
    z = jnp.maximum(y0_ref[0].astype(jnp.float32) * a0_ref[...] + d0_ref[...], 0.0)
    y1 = jax.lax.dot_general(w1_ref[...], z, (((1,), (0,)), ((), ())),
                             preferred_element_type=jnp.float32, precision=jax.lax.Precision.DEFAULT)
    y1 = y1 + b1_ref[...]
    y1_ref[0] = y1.astype(jnp.bfloat16)

    @pl.when((b == 0) & (i == 0))
    def _():
        s_ref[...] = jnp.zeros_like(s_ref)
        ss_ref[...] = jnp.zeros_like(ss_ref)
    s_ref[...] += jnp.sum(y1, axis=1, keepdims=True).T
    ss_ref[...] += jnp.sum(y1 * y1, axis=1, keepdims=True).T


def _pass3(y1_ref, a1_ref, d1_ref, out_ref):
    y1 = y1_ref[0].astype(jnp.float32)
    out_ref[0] = jnp.maximum(y1 * a1_ref[...] + d1_ref[...], 0.0)


def kernel(unknown, known, unknow_feats, known_feats,
           W0, b0, g0, beta0, W1, b1, g1, beta1):
    B, N, _ = unknown.shape
    M = known.shape[1]
    C1 = unknow_feats.shape[1]
    C2 = known_feats.shape[1]
    C0 = W0.shape[0]
    C3 = W1.shape[0]
    nblk = _NBLK if N % _NBLK == 0 else N
    nb = N // nblk
    cnt = B * N

    f32 = jnp.float32
    y0, s0, ss0 = pl.pallas_call(
        _pass1,
        grid=(B, nb),
        in_specs=[
            pl.BlockSpec((1, nblk, 3), lambda b, i: (b, i, 0)),
            pl.BlockSpec((1, 3, M), lambda b, i: (b, 0, 0)),
            pl.BlockSpec((1, C1, nblk), lambda b, i: (b, 0, i)),
            pl.BlockSpec((1, C2, M), lambda b, i: (b, 0, 0)),
            pl.BlockSpec((C0, C1 + C2), lambda b, i: (0, 0)),
            pl.BlockSpec((C0, 1), lambda b, i: (0, 0)),
        ],
        out_specs=[
            pl.BlockSpec((1, C0, nblk), lambda b, i: (b, 0, i)),
            pl.BlockSpec((1, C0), lambda b, i: (0, 0)),
            pl.BlockSpec((1, C0), lambda b, i: (0, 0)),
        ],
        out_shape=[
            jax.ShapeDtypeStruct((B, C0, N), jnp.bfloat16),
            jax.ShapeDtypeStruct((1, C0), f32),
            jax.ShapeDtypeStruct((1, C0), f32),
        ],
    )(unknown, known.transpose(0, 2, 1), unknow_feats, known_feats,
      W0, b0.reshape(C0, 1))

    mean0 = s0[0] / cnt
    var0 = ss0[0] / cnt - mean0 * mean0
    a0 = g0 / jnp.sqrt(var0 + 1e-5)
    d0 = beta0 - a0 * mean0

    y1, s1, ss1 = pl.pallas_call(
        _pass2,
        grid=(B, nb),
        in_specs=[
            pl.BlockSpec((1, C0, nblk), lambda b, i: (b, 0, i)),
            pl.BlockSpec((C3, C0), lambda b, i: (0, 0)),
            pl.BlockSpec((C3, 1), lambda b, i: (0, 0)),
            pl.BlockSpec((C0, 1), lambda b, i: (0, 0)),
            pl.BlockSpec((C0, 1), lambda b, i: (0, 0)),
        ],
        out_specs=[
            pl.BlockSpec((1, C3, nblk), lambda b, i: (b, 0, i)),
            pl.BlockSpec((1, C3), lambda b, i: (0, 0)),
            pl.BlockSpec((1, C3), lambda b, i: (0, 0)),
        ],
        out_shape=[
            jax.ShapeDtypeStruct((B, C3, N), jnp.bfloat16),
            jax.ShapeDtypeStruct((1, C3), f32),
            jax.ShapeDtypeStruct((1, C3), f32),
        ],
    )(y0, W1, b1.reshape(C3, 1), a0.reshape(C0, 1), d0.reshape(C0, 1))

    mean1 = s1[0] / cnt
    var1 = ss1[0] / cnt - mean1 * mean1
    a1 = g1 / jnp.sqrt(var1 + 1e-5)
    d1 = beta1 - a1 * mean1

    out = pl.pallas_call(
        _pass3,
        grid=(B, nb),
        in_specs=[
            pl.BlockSpec((1, C3, nblk), lambda b, i: (b, 0, i)),
            pl.BlockSpec((C3, 1), lambda b, i: (0, 0)),
            pl.BlockSpec((C3, 1), lambda b, i: (0, 0)),
        ],
        out_specs=pl.BlockSpec((1, C3, nblk), lambda b, i: (b, 0, i)),
        out_shape=jax.ShapeDtypeStruct((B, C3, N), f32),
    )(y1, a1.reshape(C3, 1), d1.reshape(C3, 1))

    return out


# speedup vs baseline: 1.9023x; 1.1373x over previous
import jax
import jax.numpy as jnp
from jax.experimental import pallas as pl

_NBLK = 2048


def _pass1(u_ref, k_ref, uf_ref, kf_ref, w0_ref, b0_ref,
           y0_ref, s_ref, ss_ref):
    b = pl.program_id(0)
    i = pl.program_id(1)

    u = u_ref[0]
    kt = k_ref[0]
    du0 = u[:, 0:1] - kt[0:1, :]
    du1 = u[:, 1:2] - kt[1:2, :]
    du2 = u[:, 2:3] - kt[2:3, :]
    d2 = du0 * du0 + du1 * du1 + du2 * du2

    inf = jnp.float32(jnp.inf)
    m1 = jnp.min(d2, axis=1, keepdims=True)
    c1 = d2 <= m1
    m2 = jnp.min(jnp.where(c1, inf, d2), axis=1, keepdims=True)
    c2 = d2 <= m2
    m3 = jnp.min(jnp.where(c2, inf, d2), axis=1, keepdims=True)
    c3 = d2 <= m3

    r1 = jax.lax.rsqrt(m1)
    r2 = jax.lax.rsqrt(m2)
    r3 = jax.lax.rsqrt(m3)
    rnorm = 1.0 / (r1 + r2 + r3)
    wint = jnp.where(c3, jax.lax.rsqrt(d2) * rnorm, 0.0)

    kf = kf_ref[0]
    C2 = kf.shape[0]
    interp = jax.lax.dot_general(kf, wint, (((1,), (1,)), ((), ())),
                                 preferred_element_type=jnp.float32, precision=jax.lax.Precision.DEFAULT)
    y0 = jax.lax.dot_general(w0_ref[:, :C2], interp, (((1,), (0,)), ((), ())),
                             preferred_element_type=jnp.float32, precision=jax.lax.Precision.DEFAULT)
    y0 = y0 + jax.lax.dot_general(w0_ref[:, C2:], uf_ref[0], (((1,), (0,)), ((), ())),
                                  preferred_element_type=jnp.float32, precision=jax.lax.Precision.DEFAULT)
    y0 = y0 + b0_ref[...]
    y0_ref[0] = y0.astype(jnp.bfloat16)

    @pl.when((b == 0) & (i == 0))
    def _():
        s_ref[...] = jnp.zeros_like(s_ref)
        ss_ref[...] = jnp.zeros_like(ss_ref)
    s_ref[...] += jnp.sum(y0, axis=1, keepdims=True).T
    ss_ref[...] += jnp.sum(y0 * y0, axis=1, keepdims=True).T


def _pass2(y0_ref, w1_ref, b1_ref, a0_ref, d0_ref,
           y1_ref, s_ref, ss_ref):
    b = pl.program_id(0)
    i = pl.program_id(1)
    z = jnp.maximum(y0_ref[0].astype(jnp.float32) * a0_ref[...] + d0_ref[...], 0.0)
    y1 = jax.lax.dot_general(w1_ref[...], z, (((1,), (0,)), ((), ())),
                             preferred_element_type=jnp.float32, precision=jax.lax.Precision.DEFAULT)
    y1 = y1 + b1_ref[...]
    y1_ref[0] = y1.astype(jnp.bfloat16)

    @pl.when((b == 0) & (i == 0))
    def _():
        s_ref[...] = jnp.zeros_like(s_ref)
        ss_ref[...] = jnp.zeros_like(ss_ref)
    s_ref[...] += jnp.sum(y1, axis=1, keepdims=True).T
    ss_ref[...] += jnp.sum(y1 * y1, axis=1, keepdims=True).T


def _pass3(y1_ref, a1_ref, d1_ref, out_ref):
    y1 = y1_ref[0].astype(jnp.float32)
    out_ref[0] = jnp.maximum(y1 * a1_ref[...] + d1_ref[...], 0.0)


def kernel(unknown, known, unknow_feats, known_feats,
           W0, b0, g0, beta0, W1, b1, g1, beta1):
    B, N, _ = unknown.shape
    M = known.shape[1]
    C1 = unknow_feats.shape[1]
    C2 = known_feats.shape[1]
    C0 = W0.shape[0]
    C3 = W1.shape[0]
    nblk = _NBLK if N % _NBLK == 0 else N
    nb = N // nblk
    cnt = B * N

    f32 = jnp.float32
    y0, s0, ss0 = pl.pallas_call(
        _pass1,
        grid=(B, nb),
        in_specs=[
            pl.BlockSpec((1, nblk, 3), lambda b, i: (b, i, 0)),
            pl.BlockSpec((1, 3, M), lambda b, i: (b, 0, 0)),
            pl.BlockSpec((1, C1, nblk), lambda b, i: (b, 0, i)),
            pl.BlockSpec((1, C2, M), lambda b, i: (b, 0, 0)),
            pl.BlockSpec((C0, C1 + C2), lambda b, i: (0, 0)),
            pl.BlockSpec((C0, 1), lambda b, i: (0, 0)),
        ],
        out_specs=[
            pl.BlockSpec((1, C0, nblk), lambda b, i: (b, 0, i)),
            pl.BlockSpec((1, C0), lambda b, i: (0, 0)),
            pl.BlockSpec((1, C0), lambda b, i: (0, 0)),
        ],
        out_shape=[
            jax.ShapeDtypeStruct((B, C0, N), jnp.bfloat16),
            jax.ShapeDtypeStruct((1, C0), f32),
            jax.ShapeDtypeStruct((1, C0), f32),
        ],
    )(unknown, known.transpose(0, 2, 1), unknow_feats, known_feats,
      W0, b0.reshape(C0, 1))

    mean0 = s0[0] / cnt
    var0 = ss0[0] / cnt - mean0 * mean0
    a0 = g0 / jnp.sqrt(var0 + 1e-5)
    d0 = beta0 - a0 * mean0

    y1, s1, ss1 = pl.pallas_call(
        _pass2,
        grid=(B, nb),
        in_specs=[
            pl.BlockSpec((1, C0, nblk), lambda b, i: (b, 0, i)),
            pl.BlockSpec((C3, C0), lambda b, i: (0, 0)),
            pl.BlockSpec((C3, 1), lambda b, i: (0, 0)),
            pl.BlockSpec((C0, 1), lambda b, i: (0, 0)),
            pl.BlockSpec((C0, 1), lambda b, i: (0, 0)),
        ],
        out_specs=[
            pl.BlockSpec((1, C3, nblk), lambda b, i: (b, 0, i)),
            pl.BlockSpec((1, C3), lambda b, i: (0, 0)),
            pl.BlockSpec((1, C3), lambda b, i: (0, 0)),
        ],
        out_shape=[
            jax.ShapeDtypeStruct((B, C3, N), jnp.bfloat16),
            jax.ShapeDtypeStruct((1, C3), f32),
            jax.ShapeDtypeStruct((1, C3), f32),
        ],
    )(y0, W1, b1.reshape(C3, 1), a0.reshape(C0, 1), d0.reshape(C0, 1))

    mean1 = s1[0] / cnt
    var1 = ss1[0] / cnt - mean1 * mean1
    a1 = g1 / jnp.sqrt(var1 + 1e-5)
    d1 = beta1 - a1 * mean1

    out = pl.pallas_call(
        _pass3,
        grid=(B, nb),
        in_specs=[
            pl.BlockSpec((1, C3, nblk), lambda b, i: (b, 0, i)),
            pl.BlockSpec((C3, 1), lambda b, i: (0, 0)),
            pl.BlockSpec((C3, 1), lambda b, i: (0, 0)),
        ],
        out_specs=pl.BlockSpec((1, C3, nblk), lambda b, i: (b, 0, i)),
        out_shape=jax.ShapeDtypeStruct((B, C3, N), f32),
    )(y1, a1.reshape(C3, 1), d1.reshape(C3, 1))

    return out


# speedup vs baseline: 2.0655x; 1.0858x over previous
import jax
import jax.numpy as jnp
from jax.experimental import pallas as pl

_NBLK = 4096


def _pass1(u_ref, k_ref, uf_ref, kf_ref, w0_ref, b0_ref,
           y0_ref, s_ref, ss_ref):
    b = pl.program_id(0)
    i = pl.program_id(1)

    u = u_ref[0]
    kt = k_ref[0]
    du0 = u[:, 0:1] - kt[0:1, :]
    du1 = u[:, 1:2] - kt[1:2, :]
    du2 = u[:, 2:3] - kt[2:3, :]
    d2 = du0 * du0 + du1 * du1 + du2 * du2

    inf = jnp.float32(jnp.inf)
    m1 = jnp.min(d2, axis=1, keepdims=True)
    c1 = d2 <= m1
    m2 = jnp.min(jnp.where(c1, inf, d2), axis=1, keepdims=True)
    c2 = d2 <= m2
    m3 = jnp.min(jnp.where(c2, inf, d2), axis=1, keepdims=True)
    c3 = d2 <= m3

    r1 = jax.lax.rsqrt(m1)
    r2 = jax.lax.rsqrt(m2)
    r3 = jax.lax.rsqrt(m3)
    rnorm = 1.0 / (r1 + r2 + r3)
    wint = jnp.where(c3, jax.lax.rsqrt(d2) * rnorm, 0.0)

    kf = kf_ref[0]
    C2 = kf.shape[0]
    interp = jax.lax.dot_general(kf, wint, (((1,), (1,)), ((), ())),
                                 preferred_element_type=jnp.float32, precision=jax.lax.Precision.DEFAULT)
    y0 = jax.lax.dot_general(w0_ref[:, :C2], interp, (((1,), (0,)), ((), ())),
                             preferred_element_type=jnp.float32, precision=jax.lax.Precision.DEFAULT)
    y0 = y0 + jax.lax.dot_general(w0_ref[:, C2:], uf_ref[0], (((1,), (0,)), ((), ())),
                                  preferred_element_type=jnp.float32, precision=jax.lax.Precision.DEFAULT)
    y0 = y0 + b0_ref[...]
    y0_ref[0] = y0.astype(jnp.bfloat16)

    @pl.when((b == 0) & (i == 0))
    def _():
        s_ref[...] = jnp.zeros_like(s_ref)
        ss_ref[...] = jnp.zeros_like(ss_ref)
    s_ref[...] += jnp.sum(y0, axis=1, keepdims=True).T
    ss_ref[...] += jnp.sum(y0 * y0, axis=1, keepdims=True).T


def _pass2(y0_ref, w1_ref, b1_ref, a0_ref, d0_ref,
           y1_ref, s_ref, ss_ref):
    b = pl.program_id(0)
    i = pl.program_id(1)
    z = jnp.maximum(y0_ref[0].astype(jnp.float32) * a0_ref[...] + d0_ref[...], 0.0)
    y1 = jax.lax.dot_general(w1_ref[...], z, (((1,), (0,)), ((), ())),
                             preferred_element_type=jnp.float32, precision=jax.lax.Precision.DEFAULT)
    y1 = y1 + b1_ref[...]
    y1_ref[0] = y1.astype(jnp.bfloat16)

    @pl.when((b == 0) & (i == 0))
    def _():
        s_ref[...] = jnp.zeros_like(s_ref)
        ss_ref[...] = jnp.zeros_like(ss_ref)
    s_ref[...] += jnp.sum(y1, axis=1, keepdims=True).T
    ss_ref[...] += jnp.sum(y1 * y1, axis=1, keepdims=True).T


def _pass3(y1_ref, a1_ref, d1_ref, out_ref):
    y1 = y1_ref[0].astype(jnp.float32)
    out_ref[0] = jnp.maximum(y1 * a1_ref[...] + d1_ref[...], 0.0)


def kernel(unknown, known, unknow_feats, known_feats,
           W0, b0, g0, beta0, W1, b1, g1, beta1):
    B, N, _ = unknown.shape
    M = known.shape[1]
    C1 = unknow_feats.shape[1]
    C2 = known_feats.shape[1]
    C0 = W0.shape[0]
    C3 = W1.shape[0]
    nblk = _NBLK if N % _NBLK == 0 else N
    nb = N // nblk
    cnt = B * N

    f32 = jnp.float32
    y0, s0, ss0 = pl.pallas_call(
        _pass1,
        grid=(B, nb),
        in_specs=[
            pl.BlockSpec((1, nblk, 3), lambda b, i: (b, i, 0)),
            pl.BlockSpec((1, 3, M), lambda b, i: (b, 0, 0)),
            pl.BlockSpec((1, C1, nblk), lambda b, i: (b, 0, i)),
            pl.BlockSpec((1, C2, M), lambda b, i: (b, 0, 0)),
            pl.BlockSpec((C0, C1 + C2), lambda b, i: (0, 0)),
            pl.BlockSpec((C0, 1), lambda b, i: (0, 0)),
        ],
        out_specs=[
            pl.BlockSpec((1, C0, nblk), lambda b, i: (b, 0, i)),
            pl.BlockSpec((1, C0), lambda b, i: (0, 0)),
            pl.BlockSpec((1, C0), lambda b, i: (0, 0)),
        ],
        out_shape=[
            jax.ShapeDtypeStruct((B, C0, N), jnp.bfloat16),
            jax.ShapeDtypeStruct((1, C0), f32),
            jax.ShapeDtypeStruct((1, C0), f32),
        ],
    )(unknown, known.transpose(0, 2, 1), unknow_feats, known_feats,
      W0, b0.reshape(C0, 1))

    mean0 = s0[0] / cnt
    var0 = ss0[0] / cnt - mean0 * mean0
    a0 = g0 / jnp.sqrt(var0 + 1e-5)
    d0 = beta0 - a0 * mean0

    y1, s1, ss1 = pl.pallas_call(
        _pass2,
        grid=(B, nb),
        in_specs=[
            pl.BlockSpec((1, C0, nblk), lambda b, i: (b, 0, i)),
            pl.BlockSpec((C3, C0), lambda b, i: (0, 0)),
            pl.BlockSpec((C3, 1), lambda b, i: (0, 0)),
            pl.BlockSpec((C0, 1), lambda b, i: (0, 0)),
            pl.BlockSpec((C0, 1), lambda b, i: (0, 0)),
        ],
        out_specs=[
            pl.BlockSpec((1, C3, nblk), lambda b, i: (b, 0, i)),
            pl.BlockSpec((1, C3), lambda b, i: (0, 0)),
            pl.BlockSpec((1, C3), lambda b, i: (0, 0)),
        ],
        out_shape=[
            jax.ShapeDtypeStruct((B, C3, N), jnp.bfloat16),
            jax.ShapeDtypeStruct((1, C3), f32),
            jax.ShapeDtypeStruct((1, C3), f32),
        ],
    )(y0, W1, b1.reshape(C3, 1), a0.reshape(C0, 1), d0.reshape(C0, 1))

    mean1 = s1[0] / cnt
    var1 = ss1[0] / cnt - mean1 * mean1
    a1 = g1 / jnp.sqrt(var1 + 1e-5)
    d1 = beta1 - a1 * mean1

    out = pl.pallas_call(
        _pass3,
        grid=(B, nb),
        in_specs=[
            pl.BlockSpec((1, C3, nblk), lambda b, i: (b, 0, i)),
            pl.BlockSpec((C3, 1), lambda b, i: (0, 0)),
            pl.BlockSpec((C3, 1), lambda b, i: (0, 0)),
        ],
        out_specs=pl.BlockSpec((1, C3, nblk), lambda b, i: (b, 0, i)),
        out_shape=jax.ShapeDtypeStruct((B, C3, N), f32),
    )(y1, a1.reshape(C3, 1), d1.reshape(C3, 1))

    return out
